# baseline (speedup 1.0000x reference)
import functools

import jax
import jax.numpy as jnp
from jax import lax
from jax.experimental import pallas as pl
from jax.experimental.pallas import tpu as pltpu
from jax.experimental.pallas import tpu_sc as plsc

B, N, P, E = 64, 4, 197, 768
POOL = 8192
PLEN = 8
K = 5
ROWS = B * N
ROW_BLK = 8
B_BLK = ROW_BLK // N
GRID = ROWS // ROW_BLK


def _sim_topk_body(x_ref, pk_ref, idx_ref):
    xb = x_ref[...].reshape(ROW_BLK, P, E)
    xm = jnp.sum(xb, axis=1) * (1.0 / P)
    sim = lax.dot_general(
        xm, pk_ref[...],
        dimension_numbers=(((1,), (1,)), ((), ())),
        preferred_element_type=jnp.float32,
    )
    iota = lax.broadcasted_iota(jnp.int32, sim.shape, 1)
    big = jnp.int32(2 ** 30)
    cols = []
    for _ in range(K):
        m = jnp.max(sim, axis=1, keepdims=True)
        cand = jnp.where(sim >= m, iota, big)
        ik = jnp.min(cand, axis=1)
        cols.append(ik[:, None])
        sim = jnp.where(iota == ik[:, None], -jnp.inf, sim)
    idx_ref[...] = jnp.concatenate(cols, axis=1)


def _sim_topk(x, prompt_key):
    return pl.pallas_call(
        _sim_topk_body,
        grid=(GRID,),
        in_specs=[
            pl.BlockSpec((B_BLK, N, P, E), lambda i: (i, 0, 0, 0)),
            pl.BlockSpec((POOL, E), lambda i: (0, 0)),
        ],
        out_specs=pl.BlockSpec((ROW_BLK, K), lambda i: (i, 0)),
        out_shape=jax.ShapeDtypeStruct((ROWS, K), jnp.int32),
    )(x, prompt_key)


_NC, _NS = 2, 16
NW = _NC * _NS
G = ROWS * K
PER_W = G // NW
CH = 8
NCH = PER_W // CH
LANES = 128


@functools.cache
def _make_gather_sc():
    mesh = plsc.VectorSubcoreMesh(core_axis_name="c", subcore_axis_name="s")

    @functools.partial(
        pl.kernel,
        mesh=mesh,
        out_type=jax.ShapeDtypeStruct((G, PLEN, E), jnp.float32),
        scratch_types=[
            pltpu.VMEM((LANES,), jnp.int32),
            pltpu.VMEM((CH, PLEN, E), jnp.float32),
            pltpu.VMEM((CH, PLEN, E), jnp.float32),
            pltpu.SemaphoreType.DMA,
            pltpu.SemaphoreType.DMA,
        ],
        compiler_params=pltpu.CompilerParams(use_tc_tiling_on_sc=True),
    )
    def _gather_sc(table_hbm, idx_hbm, out_hbm, idx_v, buf0, buf1, sem0, sem1):
        wid = lax.axis_index("s") * _NC + lax.axis_index("c")
        base = wid * PER_W
        pltpu.sync_copy(idx_hbm.at[wid], idx_v)
        bufs = (buf0, buf1)
        sems = (sem0, sem1)
        cps = [None, None]

        def start(c):
            s = c % 2
            cps[s] = pltpu.async_copy(
                table_hbm.at[idx_v.at[pl.ds(c * CH, CH)]], bufs[s], sems[s])

        start(0)
        for c in range(NCH):
            if c + 1 < NCH:
                start(c + 1)
            s = c % 2
            cps[s].wait()
            pltpu.sync_copy(bufs[s], out_hbm.at[pl.ds(base + c * CH, CH)])

    return _gather_sc


def kernel(x, prompt_key, prompt_value):
    idx = _sim_topk(x, prompt_key)
    idx_w = jnp.pad(idx.reshape(NW, PER_W), ((0, 0), (0, LANES - PER_W)))
    rows = _make_gather_sc()(prompt_value, idx_w)
    return rows.reshape(B, N, K, PLEN, E)


# speedup vs baseline: 1.4933x; 1.4933x over previous
import functools

import jax
import jax.numpy as jnp
from jax import lax
from jax.experimental import pallas as pl
from jax.experimental.pallas import tpu as pltpu
from jax.experimental.pallas import tpu_sc as plsc

B, N, P, E = 64, 4, 197, 768
POOL = 8192
PLEN = 8
K = 5
ROWS = B * N
ROW_BLK = 16
B_BLK = ROW_BLK // N
GRID = ROWS // ROW_BLK
POOL_BLK = 2048
POOL_GRID = POOL // POOL_BLK


def _mean_body(x_ref, xm_ref):
    xb = x_ref[...].reshape(ROW_BLK, P, E)
    xm_ref[...] = jnp.sum(xb, axis=1) * (1.0 / P)


def _mean(x):
    return pl.pallas_call(
        _mean_body,
        grid=(GRID,),
        in_specs=[pl.BlockSpec((B_BLK, N, P, E), lambda i: (i, 0, 0, 0))],
        out_specs=pl.BlockSpec((ROW_BLK, E), lambda i: (i, 0)),
        out_shape=jax.ShapeDtypeStruct((ROWS, E), jnp.float32),
    )(x)


def _sim_topk_body(xm_ref, pk_ref, idx_ref, sim_ref):
    j = pl.program_id(0)
    sim_ref[:, pl.ds(j * POOL_BLK, POOL_BLK)] = lax.dot_general(
        xm_ref[...], pk_ref[...],
        dimension_numbers=(((1,), (1,)), ((), ())),
        preferred_element_type=jnp.float32,
    )

    @pl.when(j == POOL_GRID - 1)
    def _topk():
        sim = sim_ref[...]
        iota = lax.broadcasted_iota(jnp.int32, sim.shape, 1)
        big = jnp.int32(2 ** 30)
        cols = []
        for _ in range(K):
            m = jnp.max(sim, axis=1, keepdims=True)
            cand = jnp.where(sim >= m, iota, big)
            ik = jnp.min(cand, axis=1)
            cols.append(ik[:, None])
            sim = jnp.where(iota == ik[:, None], -jnp.inf, sim)
        idx_ref[...] = jnp.concatenate(cols, axis=1)


def _sim_topk(xm, prompt_key):
    return pl.pallas_call(
        _sim_topk_body,
        grid=(POOL_GRID,),
        in_specs=[
            pl.BlockSpec((ROWS, E), lambda j: (0, 0)),
            pl.BlockSpec((POOL_BLK, E), lambda j: (j, 0)),
        ],
        out_specs=pl.BlockSpec((ROWS, K), lambda j: (0, 0)),
        out_shape=jax.ShapeDtypeStruct((ROWS, K), jnp.int32),
        scratch_shapes=[pltpu.VMEM((ROWS, POOL), jnp.float32)],
    )(xm, prompt_key)


_NC, _NS = 2, 16
NW = _NC * _NS
G = ROWS * K
PER_W = G // NW
CH = 8
NCH = PER_W // CH
LANES = 128


@functools.cache
def _make_gather_sc():
    mesh = plsc.VectorSubcoreMesh(core_axis_name="c", subcore_axis_name="s")

    @functools.partial(
        pl.kernel,
        mesh=mesh,
        out_type=jax.ShapeDtypeStruct((G, PLEN, E), jnp.float32),
        scratch_types=[
            pltpu.VMEM((LANES,), jnp.int32),
            pltpu.VMEM((CH, PLEN, E), jnp.float32),
            pltpu.VMEM((CH, PLEN, E), jnp.float32),
            pltpu.SemaphoreType.DMA,
            pltpu.SemaphoreType.DMA,
        ],
        compiler_params=pltpu.CompilerParams(use_tc_tiling_on_sc=True),
    )
    def _gather_sc(table_hbm, idx_hbm, out_hbm, idx_v, buf0, buf1, sem0, sem1):
        wid = lax.axis_index("s") * _NC + lax.axis_index("c")
        base = wid * PER_W
        pltpu.sync_copy(idx_hbm.at[wid], idx_v)
        bufs = (buf0, buf1)
        sems = (sem0, sem1)
        cps = [None, None]

        def start(c):
            s = c % 2
            cps[s] = pltpu.async_copy(
                table_hbm.at[idx_v.at[pl.ds(c * CH, CH)]], bufs[s], sems[s])

        start(0)
        for c in range(NCH):
            if c + 1 < NCH:
                start(c + 1)
            s = c % 2
            cps[s].wait()
            pltpu.sync_copy(bufs[s], out_hbm.at[pl.ds(base + c * CH, CH)])

    return _gather_sc


def kernel(x, prompt_key, prompt_value):
    idx = _sim_topk(_mean(x), prompt_key)
    idx_w = jnp.pad(idx.reshape(NW, PER_W), ((0, 0), (0, LANES - PER_W)))
    rows = _make_gather_sc()(prompt_value, idx_w)
    return rows.reshape(B, N, K, PLEN, E)


# speedup vs baseline: 1.4969x; 1.0024x over previous
import functools

import jax
import jax.numpy as jnp
from jax import lax
from jax.experimental import pallas as pl
from jax.experimental.pallas import tpu as pltpu
from jax.experimental.pallas import tpu_sc as plsc

B, N, P, E = 64, 4, 197, 768
POOL = 8192
PLEN = 8
K = 5
ROWS = B * N
ROW_BLK = 16
B_BLK = ROW_BLK // N
GRID = ROWS // ROW_BLK
PK_BLK = POOL // GRID
SIM_BLK = 2048


def _fused_body(x_ref, pk_ref, idx_ref, xm_ref, pk_s, sim_s):
    i = pl.program_id(0)
    xb = x_ref[...].reshape(ROW_BLK, P, E)
    xm_ref[pl.ds(i * ROW_BLK, ROW_BLK), :] = jnp.sum(xb, axis=1) * (1.0 / P)
    pk_s[pl.ds(i * PK_BLK, PK_BLK), :] = pk_ref[...]

    @pl.when(i == GRID - 1)
    def _tail():
        xm = xm_ref[...]
        for jj in range(POOL // SIM_BLK):
            sim_s[:, pl.ds(jj * SIM_BLK, SIM_BLK)] = lax.dot_general(
                xm, pk_s[pl.ds(jj * SIM_BLK, SIM_BLK), :],
                dimension_numbers=(((1,), (1,)), ((), ())),
                preferred_element_type=jnp.float32,
            )
        sim = sim_s[...]
        iota = lax.broadcasted_iota(jnp.int32, sim.shape, 1)
        big = jnp.int32(2 ** 30)
        cols = []
        for _ in range(K):
            m = jnp.max(sim, axis=1, keepdims=True)
            cand = jnp.where(sim >= m, iota, big)
            ik = jnp.min(cand, axis=1)
            cols.append(ik[:, None])
            sim = jnp.where(iota == ik[:, None], -jnp.inf, sim)
        idx_ref[...] = jnp.concatenate(cols, axis=1)


def _fused_topk(x, prompt_key):
    return pl.pallas_call(
        _fused_body,
        grid=(GRID,),
        in_specs=[
            pl.BlockSpec((B_BLK, N, P, E), lambda i: (i, 0, 0, 0)),
            pl.BlockSpec((PK_BLK, E), lambda i: (i, 0)),
        ],
        out_specs=pl.BlockSpec((ROWS, K), lambda i: (0, 0)),
        out_shape=jax.ShapeDtypeStruct((ROWS, K), jnp.int32),
        scratch_shapes=[
            pltpu.VMEM((ROWS, E), jnp.float32),
            pltpu.VMEM((POOL, E), jnp.float32),
            pltpu.VMEM((ROWS, POOL), jnp.float32),
        ],
    )(x, prompt_key)


_NC, _NS = 2, 16
NW = _NC * _NS
G = ROWS * K
PER_W = G // NW
CH = 8
NCH = PER_W // CH
LANES = 128


@functools.cache
def _make_gather_sc():
    mesh = plsc.VectorSubcoreMesh(core_axis_name="c", subcore_axis_name="s")

    @functools.partial(
        pl.kernel,
        mesh=mesh,
        out_type=jax.ShapeDtypeStruct((G, PLEN, E), jnp.float32),
        scratch_types=[
            pltpu.VMEM((LANES,), jnp.int32),
            pltpu.VMEM((CH, PLEN, E), jnp.float32),
            pltpu.VMEM((CH, PLEN, E), jnp.float32),
            pltpu.SemaphoreType.DMA,
            pltpu.SemaphoreType.DMA,
        ],
        compiler_params=pltpu.CompilerParams(use_tc_tiling_on_sc=True),
    )
    def _gather_sc(table_hbm, idx_hbm, out_hbm, idx_v, buf0, buf1, sem0, sem1):
        wid = lax.axis_index("s") * _NC + lax.axis_index("c")
        base = wid * PER_W
        pltpu.sync_copy(idx_hbm.at[wid], idx_v)
        bufs = (buf0, buf1)
        sems = (sem0, sem1)
        cps = [None, None]

        def start(c):
            s = c % 2
            cps[s] = pltpu.async_copy(
                table_hbm.at[idx_v.at[pl.ds(c * CH, CH)]], bufs[s], sems[s])

        start(0)
        for c in range(NCH):
            if c + 1 < NCH:
                start(c + 1)
            s = c % 2
            cps[s].wait()
            pltpu.sync_copy(bufs[s], out_hbm.at[pl.ds(base + c * CH, CH)])

    return _gather_sc


def kernel(x, prompt_key, prompt_value):
    idx = _fused_topk(x, prompt_key)
    idx_w = jnp.pad(idx.reshape(NW, PER_W), ((0, 0), (0, LANES - PER_W)))
    rows = _make_gather_sc()(prompt_value, idx_w)
    return rows.reshape(B, N, K, PLEN, E)
